# baseline (speedup 1.0000x reference)
import functools

import jax
import jax.numpy as jnp
from jax import lax
from jax.experimental import pallas as pl
from jax.experimental.pallas import tpu as pltpu
from jax.experimental.pallas import tpu_sc as plsc

B = 16
L = 512
T = B * L
V = 50265
D = 1024
P = 514
PAD = 1
EPS = 1e-5

NC = 2
NS = 16
NW = NC * NS
TPW = T // NW
K = 8
NCHUNK = TPW // K
LPS = L // 16


def _combine_tables(pos_emb, type_emb):
    def body(pos_ref, type_ref, out_ref):
        out_ref[...] = pos_ref[...] + type_ref[0, :][None, :]
    return pl.pallas_call(
        body,
        out_shape=jax.ShapeDtypeStruct((P, D), jnp.float32),
    )(pos_emb, type_emb)


def _sc_embed_body(ids_hbm, word_hbm, pt_hbm, lnw_hbm, lnb_hbm, out_hbm,
                   ids_v, pos_v, cnt_v, lnw_v, lnb_v,
                   wbuf0, wbuf1, pbuf0, pbuf1, obuf0, obuf1, jbuf0, jbuf1,
                   sp_pt, gw0, gw1, gp0, gp1, so0, so1, stg):
    wbufs = (wbuf0, wbuf1)
    pbufs = (pbuf0, pbuf1)
    obufs = (obuf0, obuf1)
    jbufs = (jbuf0, jbuf1)
    gws = (gw0, gw1)
    gps = (gp0, gp1)
    sos = (so0, so1)

    cid = lax.axis_index("c")
    sid = lax.axis_index("s")
    wid = sid * NC + cid
    seq = wid // 2
    half = wid % 2

    RPS = 32
    pltpu.async_copy(pt_hbm.at[pl.ds(sid * RPS, RPS)],
                     sp_pt.at[pl.ds(sid * RPS, RPS)], stg)

    @pl.when(sid == NS - 1)
    def _stage_tail():
        pltpu.async_copy(pt_hbm.at[pl.ds(NS * RPS, P - NS * RPS)],
                         sp_pt.at[pl.ds(NS * RPS, P - NS * RPS)], stg)

    pltpu.sync_copy(ids_hbm.at[pl.ds(seq * L, L)], ids_v.at[pl.ds(0, L)])
    pltpu.sync_copy(lnw_hbm, lnw_v)
    pltpu.sync_copy(lnb_hbm, lnb_v)

    def posloop(g, carry):
        vec = ids_v[pl.ds(g * 16, 16)]
        m = vec != PAD
        mi = jnp.where(m, jnp.int32(1), jnp.int32(0))
        cs = plsc.cumsum(mi) + carry
        cnt_v[pl.ds(g * 16, 16)] = cs
        pos_v[pl.ds(g * 16, 16)] = jnp.where(m, cs, jnp.int32(0)) + PAD
        return carry + jnp.sum(mi)
    lax.fori_loop(0, LPS, posloop, jnp.int32(0), unroll=False)

    half_base = half * TPW

    pltpu.make_async_copy(pt_hbm.at[pl.ds(0, RPS)],
                          sp_pt.at[pl.ds(0, RPS)], stg).wait()

    @pl.when(sid == NS - 1)
    def _stage_tail_wait():
        pltpu.make_async_copy(pt_hbm.at[pl.ds(0, P - NS * RPS)],
                              sp_pt.at[pl.ds(0, P - NS * RPS)], stg).wait()

    plsc.subcore_barrier()

    for b in range(2):
        pltpu.sync_copy(pt_hbm.at[pl.ds(PAD, 1)], pbufs[b].at[pl.ds(16, 1)])

    def issue_gathers(c, b):
        base = half_base + c * K
        pltpu.async_copy(word_hbm.at[ids_v.at[pl.ds(base, K)]], wbufs[b],
                         gws[b])
        am = jnp.maximum(base - 1, 0)
        s_c = jnp.where(base > 0, cnt_v[pl.ds(am, 16)][0], jnp.int32(0))
        slab_lo = s_c + 1
        al = pl.multiple_of(slab_lo - lax.rem(slab_lo, 8), 8)
        pltpu.async_copy(sp_pt.at[pl.ds(al, 16)],
                         pbufs[b].at[pl.ds(0, 16)], gps[b])
        pos16 = pos_v[pl.ds(base, 16)]
        ids16 = ids_v[pl.ds(base, 16)]
        jvec = pos16 - jnp.broadcast_to(al, (16,))
        jbufs[b][...] = jnp.where(ids16 == PAD, jnp.int32(16), jvec)

    for b in range(2):
        issue_gathers(jnp.int32(b), b)

    def do_chunk(c, b):
        wbuf, pbuf, obuf = wbufs[b], pbufs[b], obufs[b]
        pltpu.make_async_copy(word_hbm.at[pl.ds(0, K)], wbuf, gws[b]).wait()
        pltpu.make_async_copy(sp_pt.at[pl.ds(0, 16)],
                              pbuf.at[pl.ds(0, 16)], gps[b]).wait()
        jvec_l = jbufs[b][...]
        jts = [jvec_l[t] for t in range(K)]
        iota = lax.iota(jnp.int32, 16)

        def splat_sum(x):
            for sh in (1, 2, 4, 8):
                perm = (iota + sh) & 15
                x = x + x.at[perm].get(mode="promise_in_bounds")
            return x

        zero = jnp.zeros((16,), jnp.float32)

        acc = []
        for h in range(2):
            @plsc.parallel_loop(0, D // 16, carry=(zero,) * K)
            def p1(dd, a, h=h):
                accs = list(a)
                sl = pl.ds(dd * 16, 16)
                for tt in range(K // 2):
                    t = h * (K // 2) + tt
                    v = wbuf[t, sl] + pbuf[jts[t], sl]
                    obuf[t, sl] = v
                    accs[2 * tt] = accs[2 * tt] + v
                    accs[2 * tt + 1] = accs[2 * tt + 1] + v * v
                return tuple(accs)
            acc.extend(p1)

        stats = []
        for t in range(K):
            s, q = acc[2 * t], acc[2 * t + 1]
            mean = splat_sum(s) * (1.0 / D)
            x = splat_sum(q) * (1.0 / D) - mean * mean + EPS
            iv = plsc.bitcast(x, jnp.int32)
            iv = jnp.int32(0x5F3759DF) - (iv >> 1)
            y = plsc.bitcast(iv, jnp.float32)
            for _ in range(3):
                y = y * (1.5 - 0.5 * x * y * y)
            stats.append((y, mean * y))

        @plsc.parallel_loop(0, D // 16, step=1)
        def p2(dd):
            for u in range(1):
                sl = pl.ds((dd + u) * 16, 16)
                wv = lnw_v[sl]
                bv = lnb_v[sl]
                for t in range(K):
                    a, am = stats[t]
                    v = obuf[t, sl]
                    obuf[t, sl] = (v * a - am) * wv + bv
        row0 = seq * L + half_base + c * K
        pltpu.async_copy(obuf, out_hbm.at[pl.ds(row0, K)], sos[b])

    def pairloop(i, _):
        for b in range(2):
            c = 2 * i + b

            @pl.when(i > 0)
            def _wait_out():
                pltpu.make_async_copy(
                    obufs[b], out_hbm.at[pl.ds(0, K)], sos[b]).wait()

            do_chunk(c, b)

            @pl.when(c + 2 < NCHUNK)
            def _next_gather():
                issue_gathers(c + 2, b)
        return 0
    lax.fori_loop(0, NCHUNK // 2, pairloop, 0, unroll=False)
    for b in range(2):
        pltpu.make_async_copy(obufs[b], out_hbm.at[pl.ds(0, K)], sos[b]).wait()


_sc_embed = functools.partial(
    pl.kernel,
    out_type=jax.ShapeDtypeStruct((T, D), jnp.float32),
    mesh=plsc.VectorSubcoreMesh(core_axis_name="c", subcore_axis_name="s"),
    compiler_params=pltpu.CompilerParams(needs_layout_passes=False),
    scratch_types=[
        pltpu.VMEM((L + 16,), jnp.int32),
        pltpu.VMEM((L + 16,), jnp.int32),
        pltpu.VMEM((L + 16,), jnp.int32),
        pltpu.VMEM((D,), jnp.float32),
        pltpu.VMEM((D,), jnp.float32),
        pltpu.VMEM((K, D), jnp.float32),
        pltpu.VMEM((K, D), jnp.float32),
        pltpu.VMEM((17, D), jnp.float32),
        pltpu.VMEM((17, D), jnp.float32),
        pltpu.VMEM((K, D), jnp.float32),
        pltpu.VMEM((K, D), jnp.float32),
        pltpu.VMEM((16,), jnp.int32),
        pltpu.VMEM((16,), jnp.int32),
        pltpu.VMEM_SHARED((528, D), jnp.float32),
        pltpu.SemaphoreType.DMA,
        pltpu.SemaphoreType.DMA,
        pltpu.SemaphoreType.DMA,
        pltpu.SemaphoreType.DMA,
        pltpu.SemaphoreType.DMA,
        pltpu.SemaphoreType.DMA,
        pltpu.SemaphoreType.DMA,
    ],
)(_sc_embed_body)


def kernel(input_ids, seq_lens, position_ids, token_type_ids,
           word_emb, pos_emb, type_emb, ln_w, ln_b):
    pt = _combine_tables(pos_emb, type_emb)
    return _sc_embed(input_ids, word_emb, pt, ln_w, ln_b)


# speedup vs baseline: 1.0288x; 1.0288x over previous
import functools

import jax
import jax.numpy as jnp
from jax import lax
from jax.experimental import pallas as pl
from jax.experimental.pallas import tpu as pltpu
from jax.experimental.pallas import tpu_sc as plsc

B = 16
L = 512
T = B * L
V = 50265
D = 1024
P = 514
PAD = 1
EPS = 1e-5

NC = 2
NS = 16
NW = NC * NS
TPW = T // NW
K = 8
NCHUNK = TPW // K
LPS = L // 16


def _combine_tables(pos_emb, type_emb):
    def body(pos_ref, type_ref, out_ref):
        out_ref[...] = pos_ref[...] + type_ref[0, :][None, :]
    return pl.pallas_call(
        body,
        out_shape=jax.ShapeDtypeStruct((P, D), jnp.float32),
    )(pos_emb, type_emb)


def _sc_embed_body(ids_hbm, word_hbm, pt_hbm, lnw_hbm, lnb_hbm, out_hbm,
                   ids_v, pos_v, cnt_v, lnw_v, lnb_v,
                   wbuf0, wbuf1, pbuf0, pbuf1, obuf0, obuf1, jbuf0, jbuf1,
                   sp_pt, gw0, gw1, gp0, gp1, so0, so1, stg):
    wbufs = (wbuf0, wbuf1)
    pbufs = (pbuf0, pbuf1)
    obufs = (obuf0, obuf1)
    jbufs = (jbuf0, jbuf1)
    gws = (gw0, gw1)
    gps = (gp0, gp1)
    sos = (so0, so1)

    cid = lax.axis_index("c")
    sid = lax.axis_index("s")
    wid = sid * NC + cid
    seq = wid // 2
    half = wid % 2

    RPS = 32
    pltpu.async_copy(pt_hbm.at[pl.ds(sid * RPS, RPS)],
                     sp_pt.at[pl.ds(sid * RPS, RPS)], stg)

    @pl.when(sid == NS - 1)
    def _stage_tail():
        pltpu.async_copy(pt_hbm.at[pl.ds(NS * RPS, P - NS * RPS)],
                         sp_pt.at[pl.ds(NS * RPS, P - NS * RPS)], stg)

    pltpu.sync_copy(ids_hbm.at[pl.ds(seq * L, L)], ids_v.at[pl.ds(0, L)])
    pltpu.sync_copy(lnw_hbm, lnw_v)
    pltpu.sync_copy(lnb_hbm, lnb_v)

    def posloop(g, carry):
        vec = ids_v[pl.ds(g * 16, 16)]
        m = vec != PAD
        mi = jnp.where(m, jnp.int32(1), jnp.int32(0))
        cs = plsc.cumsum(mi) + carry
        cnt_v[pl.ds(g * 16, 16)] = cs
        pos_v[pl.ds(g * 16, 16)] = jnp.where(m, cs, jnp.int32(0)) + PAD
        return carry + jnp.sum(mi)
    lax.fori_loop(0, LPS, posloop, jnp.int32(0), unroll=False)

    half_base = half * TPW

    pltpu.make_async_copy(pt_hbm.at[pl.ds(0, RPS)],
                          sp_pt.at[pl.ds(0, RPS)], stg).wait()

    @pl.when(sid == NS - 1)
    def _stage_tail_wait():
        pltpu.make_async_copy(pt_hbm.at[pl.ds(0, P - NS * RPS)],
                              sp_pt.at[pl.ds(0, P - NS * RPS)], stg).wait()

    plsc.subcore_barrier()

    for b in range(2):
        pltpu.sync_copy(pt_hbm.at[pl.ds(PAD, 1)], pbufs[b].at[pl.ds(16, 1)])

    def issue_gathers(c, b):
        base = half_base + c * K
        pltpu.async_copy(word_hbm.at[ids_v.at[pl.ds(base, K)]], wbufs[b],
                         gws[b])
        am = jnp.maximum(base - 1, 0)
        s_c = jnp.where(base > 0, cnt_v[pl.ds(am, 16)][0], jnp.int32(0))
        slab_lo = s_c + 1
        al = pl.multiple_of(slab_lo - lax.rem(slab_lo, 8), 8)
        pltpu.async_copy(sp_pt.at[pl.ds(al, 16)],
                         pbufs[b].at[pl.ds(0, 16)], gps[b])
        pos16 = pos_v[pl.ds(base, 16)]
        ids16 = ids_v[pl.ds(base, 16)]
        jvec = pos16 - jnp.broadcast_to(al, (16,))
        jbufs[b][...] = jnp.where(ids16 == PAD, jnp.int32(16), jvec)

    for b in range(2):
        issue_gathers(jnp.int32(b), b)

    def do_chunk(c, b):
        wbuf, pbuf, obuf = wbufs[b], pbufs[b], obufs[b]
        pltpu.make_async_copy(word_hbm.at[pl.ds(0, K)], wbuf, gws[b]).wait()
        pltpu.make_async_copy(sp_pt.at[pl.ds(0, 16)],
                              pbuf.at[pl.ds(0, 16)], gps[b]).wait()
        jvec_l = jbufs[b][...]
        jts = [jvec_l[t] for t in range(K)]
        iota = lax.iota(jnp.int32, 16)

        def splat_sum(x):
            for sh in (1, 2, 4, 8):
                perm = (iota + sh) & 15
                x = x + x.at[perm].get(mode="promise_in_bounds")
            return x

        zero = jnp.zeros((16,), jnp.float32)

        @plsc.parallel_loop(0, D // 16, carry=(zero,) * (2 * K))
        def p1(dd, acc):
            accs = list(acc)
            sl = pl.ds(dd * 16, 16)
            for t in range(K):
                v = wbuf[t, sl] + pbuf[jts[t], sl]
                obuf[t, sl] = v
                accs[2 * t] = accs[2 * t] + v
                accs[2 * t + 1] = accs[2 * t + 1] + v * v
            return tuple(accs)
        acc = p1

        stats = []
        for t in range(K):
            s, q = acc[2 * t], acc[2 * t + 1]
            mean = splat_sum(s) * (1.0 / D)
            x = splat_sum(q) * (1.0 / D) - mean * mean + EPS
            iv = plsc.bitcast(x, jnp.int32)
            iv = jnp.int32(0x5F3759DF) - (iv >> 1)
            y = plsc.bitcast(iv, jnp.float32)
            for _ in range(3):
                y = y * (1.5 - 0.5 * x * y * y)
            stats.append((y, mean * y))

        @plsc.parallel_loop(0, D // 16)
        def p2(dd):
            sl = pl.ds(dd * 16, 16)
            wv = lnw_v[sl]
            bv = lnb_v[sl]
            for t in range(K):
                a, am = stats[t]
                v = obuf[t, sl]
                obuf[t, sl] = (v * a - am) * wv + bv
        row0 = seq * L + half_base + c * K
        pltpu.async_copy(obuf, out_hbm.at[pl.ds(row0, K)], sos[b])

    def pairloop(i, _):
        for b in range(2):
            c = 2 * i + b

            @pl.when(i > 0)
            def _wait_out():
                pltpu.make_async_copy(
                    obufs[b], out_hbm.at[pl.ds(0, K)], sos[b]).wait()

            do_chunk(c, b)

            @pl.when(c + 2 < NCHUNK)
            def _next_gather():
                issue_gathers(c + 2, b)
        return 0
    lax.fori_loop(0, NCHUNK // 2, pairloop, 0, unroll=False)
    for b in range(2):
        pltpu.make_async_copy(obufs[b], out_hbm.at[pl.ds(0, K)], sos[b]).wait()


_sc_embed = functools.partial(
    pl.kernel,
    out_type=jax.ShapeDtypeStruct((T, D), jnp.float32),
    mesh=plsc.VectorSubcoreMesh(core_axis_name="c", subcore_axis_name="s"),
    compiler_params=pltpu.CompilerParams(needs_layout_passes=False),
    scratch_types=[
        pltpu.VMEM((L + 16,), jnp.int32),
        pltpu.VMEM((L + 16,), jnp.int32),
        pltpu.VMEM((L + 16,), jnp.int32),
        pltpu.VMEM((D,), jnp.float32),
        pltpu.VMEM((D,), jnp.float32),
        pltpu.VMEM((K, D), jnp.float32),
        pltpu.VMEM((K, D), jnp.float32),
        pltpu.VMEM((17, D), jnp.float32),
        pltpu.VMEM((17, D), jnp.float32),
        pltpu.VMEM((K, D), jnp.float32),
        pltpu.VMEM((K, D), jnp.float32),
        pltpu.VMEM((16,), jnp.int32),
        pltpu.VMEM((16,), jnp.int32),
        pltpu.VMEM_SHARED((528, D), jnp.float32),
        pltpu.SemaphoreType.DMA,
        pltpu.SemaphoreType.DMA,
        pltpu.SemaphoreType.DMA,
        pltpu.SemaphoreType.DMA,
        pltpu.SemaphoreType.DMA,
        pltpu.SemaphoreType.DMA,
        pltpu.SemaphoreType.DMA,
    ],
)(_sc_embed_body)


def kernel(input_ids, seq_lens, position_ids, token_type_ids,
           word_emb, pos_emb, type_emb, ln_w, ln_b):
    pt = _combine_tables(pos_emb, type_emb)
    return _sc_embed(input_ids, word_emb, pt, ln_w, ln_b)
